# baseline (speedup 1.0000x reference)
import jax
from jax.experimental import pallas as pl
from jax.experimental.pallas import tpu as pltpu


def _copy_all(h_in, r_in, t_in, m_in, h_out, r_out, t_out, m_out, sems):
    pairs = ((h_in, h_out), (r_in, r_out), (t_in, t_out), (m_in, m_out))
    copies = [
        pltpu.make_async_copy(src, dst, sems.at[i])
        for i, (src, dst) in enumerate(pairs)
    ]
    for c in copies:
        c.start()
    for c in copies:
        c.wait()


def kernel(head_w, relation_w, tail_w, r_mat):
    out_shape = tuple(
        jax.ShapeDtypeStruct(x.shape, x.dtype)
        for x in (head_w, relation_w, tail_w, r_mat)
    )
    return pl.pallas_call(
        _copy_all,
        out_shape=out_shape,
        in_specs=[pl.BlockSpec(memory_space=pl.ANY)] * 4,
        out_specs=tuple(pl.BlockSpec(memory_space=pl.ANY) for _ in range(4)),
        scratch_shapes=[pltpu.SemaphoreType.DMA((4,))],
    )(head_w, relation_w, tail_w, r_mat)


# speedup vs baseline: 17.9709x; 17.9709x over previous
import jax
from jax.experimental import pallas as pl
from jax.experimental.pallas import tpu as pltpu

_GRID = 50


def _copy_body(h_in, r_in, t_in, m_in, h_out, r_out, t_out, m_out):
    h_out[...] = h_in[...]
    t_out[...] = t_in[...]
    r_out[...] = r_in[...]
    m_out[...] = m_in[...]


def kernel(head_w, relation_w, tail_w, r_mat):
    th, eh = tail_w.shape[0] // _GRID, head_w.shape[0] // _GRID
    row_spec = lambda rows, arr: pl.BlockSpec(
        (rows, arr.shape[1]), lambda i: (i, 0)
    )
    full_spec = lambda arr: pl.BlockSpec(
        arr.shape, lambda i: (0,) * arr.ndim
    )
    specs = [
        row_spec(eh, head_w),
        full_spec(relation_w),
        row_spec(th, tail_w),
        full_spec(r_mat),
    ]
    out_shape = tuple(
        jax.ShapeDtypeStruct(x.shape, x.dtype)
        for x in (head_w, relation_w, tail_w, r_mat)
    )
    return pl.pallas_call(
        _copy_body,
        grid=(_GRID,),
        in_specs=specs,
        out_specs=tuple(specs),
        out_shape=out_shape,
        compiler_params=pltpu.CompilerParams(
            dimension_semantics=("arbitrary",),
        ),
    )(head_w, relation_w, tail_w, r_mat)
